# baseline (speedup 1.0000x reference)
import functools

import jax
import jax.numpy as jnp
from jax import lax
from jax.experimental import pallas as pl
from jax.experimental.pallas import tpu as pltpu
from jax.experimental.pallas import tpu_sc as plsc

C = 16
D = H = W = 160
DHW = D * H * W
N = 524288

NC = 2
NS = 16
NW = NC * NS

_mesh = plsc.VectorSubcoreMesh(core_axis_name="c", subcore_axis_name="s")
_params = pltpu.CompilerParams(
    needs_layout_passes=False, use_tc_tiling_on_sc=False)


VPW = DHW // NW
DPW = D // NW
RB = 8
TBV = RB * W
HBLK = H // RB
NTCH = DPW * HBLK

_params_tiled = pltpu.CompilerParams(
    needs_layout_passes=False, use_tc_tiling_on_sc=True)


@functools.partial(
    pl.kernel,
    mesh=_mesh,
    compiler_params=_params_tiled,
    out_type=jax.ShapeDtypeStruct((DHW // 8, 8 * C), jnp.float32),
    scratch_types=[
        pltpu.VMEM((C, RB, W), jnp.float32),
        pltpu.VMEM((C, RB, W), jnp.float32),
        pltpu.VMEM((TBV // 8, 8 * C), jnp.float32),
        pltpu.VMEM((TBV // 8, 8 * C), jnp.float32),
        pltpu.SemaphoreType.DMA,
        pltpu.SemaphoreType.DMA,
        pltpu.SemaphoreType.DMA,
        pltpu.SemaphoreType.DMA,
    ],
)
def _to_rows(grid_hbm, table_hbm,
             in0, in1, out0, out1, si0, si1, so0, so1):
    wid = lax.axis_index("s") * NC + lax.axis_index("c")
    d_base = wid * DPW
    lanes = lax.iota(jnp.int32, 16)
    ins = (in0, in1)
    outs = (out0, out1)
    sis = (si0, si1)
    sos = (so0, so1)

    def src(ci, c):
        d = d_base + ci // HBLK
        h0 = pl.multiple_of((ci % HBLK) * RB, RB)
        return grid_hbm.at[0, c, d, pl.ds(h0, RB), :]

    def fire_in(ci, in_v, sem):
        for c in range(C):
            pltpu.async_copy(src(ci, c), in_v.at[c], sem)

    def wait_in(ci, in_v, sem):
        for c in range(C):
            pltpu.make_async_copy(src(ci, c), in_v.at[c], sem).wait()

    def dst(ci):
        v0 = pl.multiple_of((wid * VPW + ci * TBV) // 8, 8)
        return table_hbm.at[pl.ds(v0, TBV // 8)]

    def fire_out(ci, out_v, sem):
        pltpu.async_copy(out_v, dst(ci), sem)

    def wait_out(ci, out_v, sem):
        pltpu.make_async_copy(out_v, dst(ci), sem).wait()

    fire_in(0, in0, si0)

    def pair_body(ii, carry):
        for b in range(2):
            ci = ii * 2 + b

            @pl.when(ci + 1 < NTCH)
            def _():
                fire_in(ci + 1, ins[1 - b], sis[1 - b])

            wait_in(ci, ins[b], sis[b])

            @pl.when(ci >= 2)
            def _():
                wait_out(ci - 2, outs[b], sos[b])

            out_v = outs[b]
            in_v = ins[b]

            for hr in range(RB):
                @plsc.parallel_loop(0, W, unroll=8)
                def vox_body(wc, hr=hr):
                    vals = plsc.load_gather(
                        in_v,
                        [lanes, jnp.full((16,), hr, jnp.int32),
                         jnp.full((16,), wc, jnp.int32)])
                    out_v[hr * (W // 8) + (wc >> 3),
                          pl.ds((wc & 7) * 16, 16)] = vals

            fire_out(ci, outs[b], sos[b])
        return carry

    lax.fori_loop(0, NTCH // 2, pair_body, 0)
    wait_out(NTCH - 2, outs[0], sos[0])
    wait_out(NTCH - 1, outs[1], sos[1])


PPW = N // NW
CHUNK = 256
NCHUNK = PPW // CHUNK
NGRP = CHUNK // 16
SEG = 128
NSEG = CHUNK // SEG
NROW = 8 * NSEG * SEG


@functools.partial(
    pl.kernel,
    mesh=_mesh,
    compiler_params=_params,
    out_type=jax.ShapeDtypeStruct((N // 8, 8 * C), jnp.float32),
    scratch_types=[
        pltpu.VMEM((PPW,), jnp.float32),
        pltpu.VMEM((PPW,), jnp.float32),
        pltpu.VMEM((PPW,), jnp.float32),
        pltpu.VMEM((8, NSEG, SEG), jnp.int32),
        pltpu.VMEM((8, NSEG, SEG), jnp.int32),
        pltpu.VMEM((8, CHUNK), jnp.float32),
        pltpu.VMEM((8, CHUNK), jnp.float32),
        pltpu.VMEM((NROW, C), jnp.float32),
        pltpu.VMEM((NROW, C), jnp.float32),
        pltpu.VMEM((CHUNK // 8, 8 * C), jnp.float32),
        pltpu.SemaphoreType.DMA,
        pltpu.SemaphoreType.DMA,
    ],
)
def _interp(xs_hbm, ys_hbm, zs_hbm, table_hbm, out_hbm,
            cx_v, cy_v, cz_v, idx0, idx1, w0, w1, rows0, rows1, out_v,
            sem0, sem1):
    wid = lax.axis_index("s") * NC + lax.axis_index("c")
    tile_base = wid * PPW
    pltpu.sync_copy(xs_hbm.at[pl.ds(tile_base, PPW)], cx_v)
    pltpu.sync_copy(ys_hbm.at[pl.ds(tile_base, PPW)], cy_v)
    pltpu.sync_copy(zs_hbm.at[pl.ds(tile_base, PPW)], cz_v)

    bufs = ((idx0, w0, rows0, sem0), (idx1, w1, rows1, sem1))

    def compute_fire(ci, idx_v, w_v, rows_v, sem):
        @plsc.parallel_loop(0, NGRP, unroll=2)
        def grp_body(g):
            off = ci * CHUNK + g * 16
            px = cx_v[pl.ds(off, 16)]
            py = cy_v[pl.ds(off, 16)]
            pz = cz_v[pl.ds(off, 16)]
            fd = jnp.clip((px + 1.0) * (0.5 * (D - 1)), 0.0, float(D - 1))
            fh = jnp.clip((py + 1.0) * (0.5 * (H - 1)), 0.0, float(H - 1))
            fw = jnp.clip((pz + 1.0) * (0.5 * (W - 1)), 0.0, float(W - 1))
            d0 = fd.astype(jnp.int32)
            h0 = fh.astype(jnp.int32)
            w0_ = fw.astype(jnp.int32)
            wd = fd - d0.astype(jnp.float32)
            wh = fh - h0.astype(jnp.float32)
            ww = fw - w0_.astype(jnp.float32)
            d1 = jnp.minimum(d0 + 1, D - 1)
            h1 = jnp.minimum(h0 + 1, H - 1)
            w1_ = jnp.minimum(w0_ + 1, W - 1)
            ud = 1.0 - wd
            uh = 1.0 - wh
            uw = 1.0 - ww
            r00 = (d0 * H + h0) * W
            r01 = (d0 * H + h1) * W
            r10 = (d1 * H + h0) * W
            r11 = (d1 * H + h1) * W
            seg = g // (SEG // 16)
            rem = (g % (SEG // 16)) * 16
            goff = g * 16
            idxs = (r00 + w0_, r00 + w1_, r01 + w0_, r01 + w1_,
                    r10 + w0_, r10 + w1_, r11 + w0_, r11 + w1_)
            wts = (ud * uh * uw, ud * uh * ww, ud * wh * uw, ud * wh * ww,
                   wd * uh * uw, wd * uh * ww, wd * wh * uw, wd * wh * ww)
            for k in range(8):
                idx_v[k, seg, pl.ds(rem, 16)] = idxs[k]
                w_v[k, pl.ds(goff, 16)] = wts[k]

        for k in range(8):
            for s in range(NSEG):
                pltpu.async_copy(
                    table_hbm.at[idx_v.at[k, s]],
                    rows_v.at[pl.ds((k * NSEG + s) * SEG, SEG)], sem)

    def drain_combine_out(ci, idx_v, w_v, rows_v, sem):
        for k in range(8):
            for s in range(NSEG):
                pltpu.make_async_copy(
                    table_hbm.at[idx_v.at[k, s]],
                    rows_v.at[pl.ds((k * NSEG + s) * SEG, SEG)], sem).wait()

        @plsc.parallel_loop(0, NGRP, unroll=2)
        def comb_body(g):
            goff = g * 16
            lanes = lax.iota(jnp.int32, 16)
            pv = goff + lanes
            orow = (goff // 8) + (lanes // 8)
            ocol0 = (lanes % 8) * 16
            wks = [w_v[k, pl.ds(goff, 16)] for k in range(8)]
            for c in range(C):
                cv = jnp.full((16,), c, jnp.int32)
                acc = None
                for k in range(8):
                    rowv = k * (NSEG * SEG) + pv
                    vals = plsc.load_gather(rows_v, [rowv, cv])
                    acc = wks[k] * vals if acc is None else acc + wks[k] * vals
                plsc.store_scatter(out_v, [orow, ocol0 + c], acc)

        pltpu.sync_copy(
            out_v,
            out_hbm.at[pl.ds((tile_base + ci * CHUNK) // 8, CHUNK // 8)])

    compute_fire(0, *bufs[0])

    def pair_body(ii, carry):
        for b in range(2):
            ci = ii * 2 + b

            @pl.when(ci + 1 < NCHUNK)
            def _():
                compute_fire(ci + 1, *bufs[1 - b])

            drain_combine_out(ci, *bufs[b])
        return carry

    lax.fori_loop(0, NCHUNK // 2, pair_body, 0)


def kernel(xyz, grid0):
    xs = xyz[:, 0]
    ys = xyz[:, 1]
    zs = xyz[:, 2]
    table = _to_rows(grid0)
    return _interp(xs, ys, zs, table.reshape(DHW, C)).reshape(N, C)


# speedup vs baseline: 1.0004x; 1.0004x over previous
import functools

import jax
import jax.numpy as jnp
from jax import lax
from jax.experimental import pallas as pl
from jax.experimental.pallas import tpu as pltpu
from jax.experimental.pallas import tpu_sc as plsc

C = 16
D = H = W = 160
DHW = D * H * W
N = 524288

NC = 2
NS = 16
NW = NC * NS

_mesh = plsc.VectorSubcoreMesh(core_axis_name="c", subcore_axis_name="s")
_params = pltpu.CompilerParams(
    needs_layout_passes=False, use_tc_tiling_on_sc=False)


VPW = DHW // NW
DPW = D // NW
RB = 8
TBV = RB * W
HBLK = H // RB
NTCH = DPW * HBLK

_params_tiled = pltpu.CompilerParams(
    needs_layout_passes=False, use_tc_tiling_on_sc=True)


@functools.partial(
    pl.kernel,
    mesh=_mesh,
    compiler_params=_params_tiled,
    out_type=jax.ShapeDtypeStruct((DHW // 8, 8 * C), jnp.float32),
    scratch_types=[
        pltpu.VMEM((C, RB, W), jnp.float32),
        pltpu.VMEM((C, RB, W), jnp.float32),
        pltpu.VMEM((TBV // 8, 8 * C), jnp.float32),
        pltpu.VMEM((TBV // 8, 8 * C), jnp.float32),
        pltpu.SemaphoreType.DMA,
        pltpu.SemaphoreType.DMA,
        pltpu.SemaphoreType.DMA,
        pltpu.SemaphoreType.DMA,
    ],
)
def _to_rows(grid_hbm, table_hbm,
             in0, in1, out0, out1, si0, si1, so0, so1):
    wid = lax.axis_index("s") * NC + lax.axis_index("c")
    d_base = wid * DPW
    lanes = lax.iota(jnp.int32, 16)
    ins = (in0, in1)
    outs = (out0, out1)
    sis = (si0, si1)
    sos = (so0, so1)

    def src(ci, c):
        d = d_base + ci // HBLK
        h0 = pl.multiple_of((ci % HBLK) * RB, RB)
        return grid_hbm.at[0, c, d, pl.ds(h0, RB), :]

    def fire_in(ci, in_v, sem):
        for c in range(C):
            pltpu.async_copy(src(ci, c), in_v.at[c], sem)

    def wait_in(ci, in_v, sem):
        for c in range(C):
            pltpu.make_async_copy(src(ci, c), in_v.at[c], sem).wait()

    def dst(ci):
        v0 = pl.multiple_of((wid * VPW + ci * TBV) // 8, 8)
        return table_hbm.at[pl.ds(v0, TBV // 8)]

    def fire_out(ci, out_v, sem):
        pltpu.async_copy(out_v, dst(ci), sem)

    def wait_out(ci, out_v, sem):
        pltpu.make_async_copy(out_v, dst(ci), sem).wait()

    fire_in(0, in0, si0)

    def pair_body(ii, carry):
        for b in range(2):
            ci = ii * 2 + b

            @pl.when(ci + 1 < NTCH)
            def _():
                fire_in(ci + 1, ins[1 - b], sis[1 - b])

            wait_in(ci, ins[b], sis[b])

            @pl.when(ci >= 2)
            def _():
                wait_out(ci - 2, outs[b], sos[b])

            out_v = outs[b]
            in_v = ins[b]

            zero = jnp.zeros((16,), jnp.int32)

            @plsc.parallel_loop(0, TBV, unroll=8)
            def vox_body(v):
                vals = plsc.load_gather(
                    in_v, [lanes, zero, jnp.full((16,), v, jnp.int32)])
                out_v[v >> 3, pl.ds((v & 7) * 16, 16)] = vals

            fire_out(ci, outs[b], sos[b])
        return carry

    lax.fori_loop(0, NTCH // 2, pair_body, 0)
    wait_out(NTCH - 2, outs[0], sos[0])
    wait_out(NTCH - 1, outs[1], sos[1])


PPW = N // NW
CHUNK = 256
NCHUNK = PPW // CHUNK
NGRP = CHUNK // 16
SEG = 128
NSEG = CHUNK // SEG
NROW = 8 * NSEG * SEG


@functools.partial(
    pl.kernel,
    mesh=_mesh,
    compiler_params=_params,
    out_type=jax.ShapeDtypeStruct((N // 8, 8 * C), jnp.float32),
    scratch_types=[
        pltpu.VMEM((PPW,), jnp.float32),
        pltpu.VMEM((PPW,), jnp.float32),
        pltpu.VMEM((PPW,), jnp.float32),
        pltpu.VMEM((8, NSEG, SEG), jnp.int32),
        pltpu.VMEM((8, NSEG, SEG), jnp.int32),
        pltpu.VMEM((8, CHUNK), jnp.float32),
        pltpu.VMEM((8, CHUNK), jnp.float32),
        pltpu.VMEM((NROW, C), jnp.float32),
        pltpu.VMEM((NROW, C), jnp.float32),
        pltpu.VMEM((CHUNK // 8, 8 * C), jnp.float32),
        pltpu.SemaphoreType.DMA,
        pltpu.SemaphoreType.DMA,
    ],
)
def _interp(xs_hbm, ys_hbm, zs_hbm, table_hbm, out_hbm,
            cx_v, cy_v, cz_v, idx0, idx1, w0, w1, rows0, rows1, out_v,
            sem0, sem1):
    wid = lax.axis_index("s") * NC + lax.axis_index("c")
    tile_base = wid * PPW
    pltpu.sync_copy(xs_hbm.at[pl.ds(tile_base, PPW)], cx_v)
    pltpu.sync_copy(ys_hbm.at[pl.ds(tile_base, PPW)], cy_v)
    pltpu.sync_copy(zs_hbm.at[pl.ds(tile_base, PPW)], cz_v)

    bufs = ((idx0, w0, rows0, sem0), (idx1, w1, rows1, sem1))

    def compute_fire(ci, idx_v, w_v, rows_v, sem):
        @plsc.parallel_loop(0, NGRP, unroll=2)
        def grp_body(g):
            off = ci * CHUNK + g * 16
            px = cx_v[pl.ds(off, 16)]
            py = cy_v[pl.ds(off, 16)]
            pz = cz_v[pl.ds(off, 16)]
            fd = jnp.clip((px + 1.0) * (0.5 * (D - 1)), 0.0, float(D - 1))
            fh = jnp.clip((py + 1.0) * (0.5 * (H - 1)), 0.0, float(H - 1))
            fw = jnp.clip((pz + 1.0) * (0.5 * (W - 1)), 0.0, float(W - 1))
            d0 = fd.astype(jnp.int32)
            h0 = fh.astype(jnp.int32)
            w0_ = fw.astype(jnp.int32)
            wd = fd - d0.astype(jnp.float32)
            wh = fh - h0.astype(jnp.float32)
            ww = fw - w0_.astype(jnp.float32)
            d1 = jnp.minimum(d0 + 1, D - 1)
            h1 = jnp.minimum(h0 + 1, H - 1)
            w1_ = jnp.minimum(w0_ + 1, W - 1)
            ud = 1.0 - wd
            uh = 1.0 - wh
            uw = 1.0 - ww
            r00 = (d0 * H + h0) * W
            r01 = (d0 * H + h1) * W
            r10 = (d1 * H + h0) * W
            r11 = (d1 * H + h1) * W
            seg = g // (SEG // 16)
            rem = (g % (SEG // 16)) * 16
            goff = g * 16
            idxs = (r00 + w0_, r00 + w1_, r01 + w0_, r01 + w1_,
                    r10 + w0_, r10 + w1_, r11 + w0_, r11 + w1_)
            wts = (ud * uh * uw, ud * uh * ww, ud * wh * uw, ud * wh * ww,
                   wd * uh * uw, wd * uh * ww, wd * wh * uw, wd * wh * ww)
            for k in range(8):
                idx_v[k, seg, pl.ds(rem, 16)] = idxs[k]
                w_v[k, pl.ds(goff, 16)] = wts[k]

        for k in range(8):
            for s in range(NSEG):
                pltpu.async_copy(
                    table_hbm.at[idx_v.at[k, s]],
                    rows_v.at[pl.ds((k * NSEG + s) * SEG, SEG)], sem)

    def drain_combine_out(ci, idx_v, w_v, rows_v, sem):
        for k in range(8):
            for s in range(NSEG):
                pltpu.make_async_copy(
                    table_hbm.at[idx_v.at[k, s]],
                    rows_v.at[pl.ds((k * NSEG + s) * SEG, SEG)], sem).wait()

        @plsc.parallel_loop(0, NGRP, unroll=2)
        def comb_body(g):
            goff = g * 16
            lanes = lax.iota(jnp.int32, 16)
            pv = goff + lanes
            orow = (goff // 8) + (lanes // 8)
            ocol0 = (lanes % 8) * 16
            wks = [w_v[k, pl.ds(goff, 16)] for k in range(8)]
            for c in range(C):
                cv = jnp.full((16,), c, jnp.int32)
                acc = None
                for k in range(8):
                    rowv = k * (NSEG * SEG) + pv
                    vals = plsc.load_gather(rows_v, [rowv, cv])
                    acc = wks[k] * vals if acc is None else acc + wks[k] * vals
                plsc.store_scatter(out_v, [orow, ocol0 + c], acc)

        pltpu.sync_copy(
            out_v,
            out_hbm.at[pl.ds((tile_base + ci * CHUNK) // 8, CHUNK // 8)])

    compute_fire(0, *bufs[0])

    def pair_body(ii, carry):
        for b in range(2):
            ci = ii * 2 + b

            @pl.when(ci + 1 < NCHUNK)
            def _():
                compute_fire(ci + 1, *bufs[1 - b])

            drain_combine_out(ci, *bufs[b])
        return carry

    lax.fori_loop(0, NCHUNK // 2, pair_body, 0)


def kernel(xyz, grid0):
    xs = xyz[:, 0]
    ys = xyz[:, 1]
    zs = xyz[:, 2]
    table = _to_rows(grid0)
    return _interp(xs, ys, zs, table.reshape(DHW, C)).reshape(N, C)


# speedup vs baseline: 1.0989x; 1.0985x over previous
import functools

import jax
import jax.numpy as jnp
from jax import lax
from jax.experimental import pallas as pl
from jax.experimental.pallas import tpu as pltpu
from jax.experimental.pallas import tpu_sc as plsc

C = 16
D = H = W = 160
DHW = D * H * W
N = 524288

NC = 2
NS = 16
NW = NC * NS

_mesh = plsc.VectorSubcoreMesh(core_axis_name="c", subcore_axis_name="s")
_params = pltpu.CompilerParams(
    needs_layout_passes=False, use_tc_tiling_on_sc=False)


VPW = DHW // NW
TB = 2000
NTCH = VPW // TB


@functools.partial(
    pl.kernel,
    mesh=_mesh,
    compiler_params=_params,
    out_type=jax.ShapeDtypeStruct((DHW, C), jnp.float32),
    scratch_types=[
        pltpu.VMEM((C, TB), jnp.float32),
        pltpu.VMEM((C, TB), jnp.float32),
        pltpu.VMEM((TB, C), jnp.float32),
        pltpu.VMEM((TB, C), jnp.float32),
        pltpu.SemaphoreType.DMA,
        pltpu.SemaphoreType.DMA,
        pltpu.SemaphoreType.DMA,
        pltpu.SemaphoreType.DMA,
    ],
)
def _to_rows(gflat_hbm, table_hbm,
             in0, in1, out0, out1, si0, si1, so0, so1):
    wid = lax.axis_index("s") * NC + lax.axis_index("c")
    tile_base = wid * VPW
    lanes = lax.iota(jnp.int32, 16)
    ins = (in0, in1)
    outs = (out0, out1)
    sis = (si0, si1)
    sos = (so0, so1)

    def fire_in(ci, in_v, sem):
        v0 = tile_base + ci * TB
        for c in range(C):
            pltpu.async_copy(gflat_hbm.at[c, pl.ds(v0, TB)], in_v.at[c], sem)

    def wait_in(ci, in_v, sem):
        v0 = tile_base + ci * TB
        for c in range(C):
            pltpu.make_async_copy(
                gflat_hbm.at[c, pl.ds(v0, TB)], in_v.at[c], sem).wait()

    def fire_out(ci, out_v, sem):
        v0 = tile_base + ci * TB
        pltpu.async_copy(out_v, table_hbm.at[pl.ds(v0, TB)], sem)

    def wait_out(ci, out_v, sem):
        v0 = tile_base + ci * TB
        pltpu.make_async_copy(out_v, table_hbm.at[pl.ds(v0, TB)], sem).wait()

    fire_in(0, in0, si0)

    def pair_body(ii, carry):
        for b in range(2):
            ci = ii * 2 + b

            @pl.when(ci + 1 < NTCH)
            def _():
                fire_in(ci + 1, ins[1 - b], sis[1 - b])

            wait_in(ci, ins[b], sis[b])

            @pl.when(ci >= 2)
            def _():
                wait_out(ci - 2, outs[b], sos[b])

            out_v = outs[b]

            @plsc.parallel_loop(0, TB, unroll=8)
            def vox_body(v):
                vals = plsc.load_gather(
                    ins[b], [lanes, jnp.full((16,), v, jnp.int32)])
                out_v[v, :] = vals

            fire_out(ci, outs[b], sos[b])
        return carry

    lax.fori_loop(0, NTCH // 2, pair_body, 0)
    wait_out(NTCH - 2, outs[0], sos[0])
    wait_out(NTCH - 1, outs[1], sos[1])


PPW = N // NW
CHUNK = 256
NCHUNK = PPW // CHUNK
NGRP = CHUNK // 16
SEG = 128
NSEG = CHUNK // SEG
NROW = 8 * NSEG * SEG


@functools.partial(
    pl.kernel,
    mesh=_mesh,
    compiler_params=_params,
    out_type=jax.ShapeDtypeStruct((N * C,), jnp.float32),
    scratch_types=[
        pltpu.VMEM((PPW,), jnp.float32),
        pltpu.VMEM((PPW,), jnp.float32),
        pltpu.VMEM((PPW,), jnp.float32),
        pltpu.VMEM((8, NSEG, SEG), jnp.int32),
        pltpu.VMEM((8, NSEG, SEG), jnp.int32),
        pltpu.VMEM((8, CHUNK), jnp.float32),
        pltpu.VMEM((8, CHUNK), jnp.float32),
        pltpu.VMEM((NROW, C), jnp.float32),
        pltpu.VMEM((NROW, C), jnp.float32),
        pltpu.VMEM((CHUNK * C,), jnp.float32),
        pltpu.SemaphoreType.DMA,
        pltpu.SemaphoreType.DMA,
    ],
)
def _interp(xs_hbm, ys_hbm, zs_hbm, table_hbm, out_hbm,
            cx_v, cy_v, cz_v, idx0, idx1, w0, w1, rows0, rows1, out_v,
            sem0, sem1):
    wid = lax.axis_index("s") * NC + lax.axis_index("c")
    tile_base = wid * PPW
    pltpu.sync_copy(xs_hbm.at[pl.ds(tile_base, PPW)], cx_v)
    pltpu.sync_copy(ys_hbm.at[pl.ds(tile_base, PPW)], cy_v)
    pltpu.sync_copy(zs_hbm.at[pl.ds(tile_base, PPW)], cz_v)

    bufs = ((idx0, w0, rows0, sem0), (idx1, w1, rows1, sem1))

    def compute_fire(ci, idx_v, w_v, rows_v, sem):
        @plsc.parallel_loop(0, NGRP, unroll=2)
        def grp_body(g):
            off = ci * CHUNK + g * 16
            px = cx_v[pl.ds(off, 16)]
            py = cy_v[pl.ds(off, 16)]
            pz = cz_v[pl.ds(off, 16)]
            fd = jnp.clip((px + 1.0) * (0.5 * (D - 1)), 0.0, float(D - 1))
            fh = jnp.clip((py + 1.0) * (0.5 * (H - 1)), 0.0, float(H - 1))
            fw = jnp.clip((pz + 1.0) * (0.5 * (W - 1)), 0.0, float(W - 1))
            d0 = fd.astype(jnp.int32)
            h0 = fh.astype(jnp.int32)
            w0_ = fw.astype(jnp.int32)
            wd = fd - d0.astype(jnp.float32)
            wh = fh - h0.astype(jnp.float32)
            ww = fw - w0_.astype(jnp.float32)
            d1 = jnp.minimum(d0 + 1, D - 1)
            h1 = jnp.minimum(h0 + 1, H - 1)
            w1_ = jnp.minimum(w0_ + 1, W - 1)
            ud = 1.0 - wd
            uh = 1.0 - wh
            uw = 1.0 - ww
            r00 = (d0 * H + h0) * W
            r01 = (d0 * H + h1) * W
            r10 = (d1 * H + h0) * W
            r11 = (d1 * H + h1) * W
            seg = g // (SEG // 16)
            rem = (g % (SEG // 16)) * 16
            goff = g * 16
            idxs = (r00 + w0_, r00 + w1_, r01 + w0_, r01 + w1_,
                    r10 + w0_, r10 + w1_, r11 + w0_, r11 + w1_)
            wts = (ud * uh * uw, ud * uh * ww, ud * wh * uw, ud * wh * ww,
                   wd * uh * uw, wd * uh * ww, wd * wh * uw, wd * wh * ww)
            for k in range(8):
                idx_v[k, seg, pl.ds(rem, 16)] = idxs[k]
                w_v[k, pl.ds(goff, 16)] = wts[k]

        for k in range(8):
            for s in range(NSEG):
                pltpu.async_copy(
                    table_hbm.at[idx_v.at[k, s]],
                    rows_v.at[pl.ds((k * NSEG + s) * SEG, SEG)], sem)

    def drain_combine_out(ci, idx_v, w_v, rows_v, sem):
        for k in range(8):
            for s in range(NSEG):
                pltpu.make_async_copy(
                    table_hbm.at[idx_v.at[k, s]],
                    rows_v.at[pl.ds((k * NSEG + s) * SEG, SEG)], sem).wait()

        @plsc.parallel_loop(0, NGRP, unroll=2)
        def comb_body(g):
            goff = g * 16
            lanes = lax.iota(jnp.int32, 16)
            pv = goff + lanes
            pvz = pv * C
            wks = [w_v[k, pl.ds(goff, 16)] for k in range(8)]
            for c in range(C):
                cv = jnp.full((16,), c, jnp.int32)
                acc = None
                for k in range(8):
                    rowv = k * (NSEG * SEG) + pv
                    vals = plsc.load_gather(rows_v, [rowv, cv])
                    acc = wks[k] * vals if acc is None else acc + wks[k] * vals
                plsc.store_scatter(out_v, [pvz + c], acc)

        pltpu.sync_copy(
            out_v,
            out_hbm.at[pl.ds((tile_base + ci * CHUNK) * C, CHUNK * C)])

    compute_fire(0, *bufs[0])

    def pair_body(ii, carry):
        for b in range(2):
            ci = ii * 2 + b

            @pl.when(ci + 1 < NCHUNK)
            def _():
                compute_fire(ci + 1, *bufs[1 - b])

            drain_combine_out(ci, *bufs[b])
        return carry

    lax.fori_loop(0, NCHUNK // 2, pair_body, 0)


def kernel(xyz, grid0):
    xs = xyz[:, 0]
    ys = xyz[:, 1]
    zs = xyz[:, 2]
    table = _to_rows(grid0[0].reshape(C, DHW))
    return _interp(xs, ys, zs, table).reshape(N, C)
